# baseline (device time: 282202 ns/iter reference)
import jax
import jax.numpy as jnp
from jax import lax
from jax.experimental import pallas as pl
from jax.experimental.pallas import tpu as pltpu

N_DEV = 16
D_MODEL = 512
D_HIDDEN = 4096
CHUNK = D_HIDDEN // N_DEV
HQ = 64
DH = 64
N_BLK = 4
BLK = 64


def _allgather_weights(Wq_shard, Wo_shard):

    def body(wq_ref, wo_ref, wq_out, wo_out, send_q, recv_q, send_o, recv_o):
        my = lax.axis_index("i")
        left = lax.rem(my - 1 + N_DEV, N_DEV)
        right = lax.rem(my + 1, N_DEV)

        barrier_sem = pltpu.get_barrier_semaphore()
        for nbr in [left, right]:
            pl.semaphore_signal(
                barrier_sem, inc=1,
                device_id=(nbr,), device_id_type=pl.DeviceIdType.MESH,
            )
        pl.semaphore_wait(barrier_sem, 2)

        wq_out[:, pl.ds(my * CHUNK, CHUNK)] = wq_ref[...]
        wo_out[pl.ds(my * CHUNK, CHUNK), :] = wo_ref[...]

        for h in range(N_DEV - 1):
            send_origin = lax.rem(my - h + N_DEV, N_DEV)
            rq = pltpu.make_async_remote_copy(
                src_ref=wq_out.at[:, pl.ds(send_origin * CHUNK, CHUNK)],
                dst_ref=wq_out.at[:, pl.ds(send_origin * CHUNK, CHUNK)],
                send_sem=send_q.at[h],
                recv_sem=recv_q.at[h],
                device_id=(right,),
                device_id_type=pl.DeviceIdType.MESH,
            )
            ro = pltpu.make_async_remote_copy(
                src_ref=wo_out.at[pl.ds(send_origin * CHUNK, CHUNK), :],
                dst_ref=wo_out.at[pl.ds(send_origin * CHUNK, CHUNK), :],
                send_sem=send_o.at[h],
                recv_sem=recv_o.at[h],
                device_id=(right,),
                device_id_type=pl.DeviceIdType.MESH,
            )
            rq.start()
            ro.start()
            rq.wait()
            ro.wait()

    return pl.pallas_call(
        body,
        out_shape=(
            jax.ShapeDtypeStruct((D_MODEL, D_HIDDEN), jnp.float32),
            jax.ShapeDtypeStruct((D_HIDDEN, D_MODEL), jnp.float32),
        ),
        in_specs=[
            pl.BlockSpec(memory_space=pltpu.VMEM),
            pl.BlockSpec(memory_space=pltpu.VMEM),
        ],
        out_specs=(
            pl.BlockSpec(memory_space=pltpu.VMEM),
            pl.BlockSpec(memory_space=pltpu.VMEM),
        ),
        scratch_shapes=[
            pltpu.SemaphoreType.DMA((N_DEV - 1,)),
            pltpu.SemaphoreType.DMA((N_DEV - 1,)),
            pltpu.SemaphoreType.DMA((N_DEV - 1,)),
            pltpu.SemaphoreType.DMA((N_DEV - 1,)),
        ],
        compiler_params=pltpu.CompilerParams(collective_id=0),
    )(Wq_shard, Wo_shard)


def kernel(x, Wq, K_ext, V_ext, Wo):
    my = lax.axis_index("i")
    b = x.shape[0]

    Wq_full, Wo_full = _allgather_weights(Wq, Wo)

    K_loc = lax.dynamic_slice_in_dim(K_ext, my * b, b, axis=0)
    V_loc = lax.dynamic_slice_in_dim(V_ext, my * b, b, axis=0)

    Q = (x @ Wq_full).reshape(b, N_BLK, BLK, HQ, DH)
    Kb = K_loc.reshape(b, N_BLK, BLK, HQ, DH)
    Vb = V_loc.reshape(b, N_BLK, BLK, HQ, DH)

    scores = jnp.einsum("bqihd,bqjhd->bqhij", Q, Kb) * 0.125
    m = scores.max(axis=-1, keepdims=True)
    w = jnp.exp(scores - m)
    w = w / w.sum(axis=-1, keepdims=True)
    ctx = jnp.einsum("bqhij,bqjhd->bqihd", w, Vb)
    ctx = ctx.reshape(b, N_BLK * BLK, D_HIDDEN)
    return ctx @ Wo_full


# device time: 153075 ns/iter; 1.8436x vs baseline; 1.8436x over previous
import jax
import jax.numpy as jnp
from jax import lax
from jax.experimental import pallas as pl
from jax.experimental.pallas import tpu as pltpu

N_DEV = 16
D_MODEL = 512
D_HIDDEN = 4096
CHUNK = D_HIDDEN // N_DEV
HQ = 64
DH = 64
N_BLK = 4
BLK = 64

N_CW = N_DEV // 2
N_CCW = N_DEV - 1 - N_CW


def _allgather_weights(Wq_shard, Wo_shard):

    def body(wq_ref, wo_ref, wq_out, wo_out,
             cw_send_q, cw_recv_q, cw_send_o, cw_recv_o,
             ccw_send_q, ccw_recv_q, ccw_send_o, ccw_recv_o):
        my = lax.axis_index("i")
        left = lax.rem(my - 1 + N_DEV, N_DEV)
        right = lax.rem(my + 1, N_DEV)

        barrier_sem = pltpu.get_barrier_semaphore()
        for nbr in [left, right]:
            pl.semaphore_signal(
                barrier_sem, inc=1,
                device_id=(nbr,), device_id_type=pl.DeviceIdType.MESH,
            )
        pl.semaphore_wait(barrier_sem, 2)

        wq_out[:, pl.ds(my * CHUNK, CHUNK)] = wq_ref[...]
        wo_out[pl.ds(my * CHUNK, CHUNK), :] = wo_ref[...]

        def chunk_rdmas(origin, target, sq, so):
            rq = pltpu.make_async_remote_copy(
                src_ref=wq_out.at[:, pl.ds(origin * CHUNK, CHUNK)],
                dst_ref=wq_out.at[:, pl.ds(origin * CHUNK, CHUNK)],
                send_sem=sq[0], recv_sem=sq[1],
                device_id=(target,), device_id_type=pl.DeviceIdType.MESH,
            )
            ro = pltpu.make_async_remote_copy(
                src_ref=wo_out.at[pl.ds(origin * CHUNK, CHUNK), :],
                dst_ref=wo_out.at[pl.ds(origin * CHUNK, CHUNK), :],
                send_sem=so[0], recv_sem=so[1],
                device_id=(target,), device_id_type=pl.DeviceIdType.MESH,
            )
            return rq, ro

        for h in range(N_CW):
            cw = chunk_rdmas(
                lax.rem(my - h + N_DEV, N_DEV), right,
                (cw_send_q.at[h], cw_recv_q.at[h]),
                (cw_send_o.at[h], cw_recv_o.at[h]),
            )
            ccw = None
            if h < N_CCW:
                ccw = chunk_rdmas(
                    lax.rem(my + h, N_DEV), left,
                    (ccw_send_q.at[h], ccw_recv_q.at[h]),
                    (ccw_send_o.at[h], ccw_recv_o.at[h]),
                )
            for r in cw:
                r.start()
            if ccw is not None:
                for r in ccw:
                    r.start()
            for r in cw:
                r.wait()
            if ccw is not None:
                for r in ccw:
                    r.wait()

    return pl.pallas_call(
        body,
        out_shape=(
            jax.ShapeDtypeStruct((D_MODEL, D_HIDDEN), jnp.bfloat16),
            jax.ShapeDtypeStruct((D_HIDDEN, D_MODEL), jnp.bfloat16),
        ),
        in_specs=[
            pl.BlockSpec(memory_space=pltpu.VMEM),
            pl.BlockSpec(memory_space=pltpu.VMEM),
        ],
        out_specs=(
            pl.BlockSpec(memory_space=pltpu.VMEM),
            pl.BlockSpec(memory_space=pltpu.VMEM),
        ),
        scratch_shapes=[
            pltpu.SemaphoreType.DMA((N_CW,)),
            pltpu.SemaphoreType.DMA((N_CW,)),
            pltpu.SemaphoreType.DMA((N_CW,)),
            pltpu.SemaphoreType.DMA((N_CW,)),
            pltpu.SemaphoreType.DMA((N_CCW,)),
            pltpu.SemaphoreType.DMA((N_CCW,)),
            pltpu.SemaphoreType.DMA((N_CCW,)),
            pltpu.SemaphoreType.DMA((N_CCW,)),
        ],
        compiler_params=pltpu.CompilerParams(collective_id=0),
    )(Wq_shard, Wo_shard)


def kernel(x, Wq, K_ext, V_ext, Wo):
    my = lax.axis_index("i")
    b = x.shape[0]

    Wq_full, Wo_full = _allgather_weights(
        Wq.astype(jnp.bfloat16), Wo.astype(jnp.bfloat16)
    )
    Wq_full = Wq_full.astype(jnp.float32)
    Wo_full = Wo_full.astype(jnp.float32)

    K_loc = lax.dynamic_slice_in_dim(K_ext, my * b, b, axis=0)
    V_loc = lax.dynamic_slice_in_dim(V_ext, my * b, b, axis=0)

    Q = (x @ Wq_full).reshape(b, N_BLK, BLK, HQ, DH)
    Kb = K_loc.reshape(b, N_BLK, BLK, HQ, DH)
    Vb = V_loc.reshape(b, N_BLK, BLK, HQ, DH)

    scores = jnp.einsum("bqihd,bqjhd->bqhij", Q, Kb) * 0.125
    m = scores.max(axis=-1, keepdims=True)
    w = jnp.exp(scores - m)
    w = w / w.sum(axis=-1, keepdims=True)
    ctx = jnp.einsum("bqhij,bqjhd->bqihd", w, Vb)
    ctx = ctx.reshape(b, N_BLK * BLK, D_HIDDEN)
    return ctx @ Wo_full


# device time: 83992 ns/iter; 3.3599x vs baseline; 1.8225x over previous
import jax
import jax.numpy as jnp
from jax import lax
from jax.experimental import pallas as pl
from jax.experimental.pallas import tpu as pltpu

N_DEV = 16
D_MODEL = 512
D_HIDDEN = 4096
CHUNK = D_HIDDEN // N_DEV
HQ = 64
DH = 64
H_PER = 4
SQ = 256
BLK = 64

N_CW = N_DEV // 2
N_CCW = N_DEV - 1 - N_CW

F32 = jnp.float32
BF16 = jnp.bfloat16


def _fused(x_bf, wq_sh, wo_sh, k_bf, v_bf):
    n_b = x_bf.shape[0]

    def body(x_ref, wq_ref, wo_ref, k_ref, v_ref, out_ref,
             wq_buf, wo_buf, ctx_buf,
             cw_send_q, cw_recv_q, cw_send_o, cw_recv_o,
             ccw_send_q, ccw_recv_q, ccw_send_o, ccw_recv_o):
        my = lax.axis_index("i")
        left = lax.rem(my - 1 + N_DEV, N_DEV)
        right = lax.rem(my + 1, N_DEV)

        qi = lax.broadcasted_iota(jnp.int32, (SQ, SQ), 0) // BLK
        kj = lax.broadcasted_iota(jnp.int32, (SQ, SQ), 1) // BLK
        neg = jnp.where(qi == kj, 0.0, -1e9).astype(F32)

        barrier_sem = pltpu.get_barrier_semaphore()
        for nbr in [left, right]:
            pl.semaphore_signal(
                barrier_sem, inc=1,
                device_id=(nbr,), device_id_type=pl.DeviceIdType.MESH,
            )
        pl.semaphore_wait(barrier_sem, 2)

        wq_buf[:, pl.ds(my * CHUNK, CHUNK)] = wq_ref[...]
        wo_buf[pl.ds(my * CHUNK, CHUNK), :] = wo_ref[...]

        def send_chunk(origin, target, sq_send, sq_recv, so_send, so_recv):
            rq = pltpu.make_async_remote_copy(
                src_ref=wq_buf.at[:, pl.ds(origin * CHUNK, CHUNK)],
                dst_ref=wq_buf.at[:, pl.ds(origin * CHUNK, CHUNK)],
                send_sem=sq_send, recv_sem=sq_recv,
                device_id=(target,), device_id_type=pl.DeviceIdType.MESH,
            )
            ro = pltpu.make_async_remote_copy(
                src_ref=wo_buf.at[pl.ds(origin * CHUNK, CHUNK), :],
                dst_ref=wo_buf.at[pl.ds(origin * CHUNK, CHUNK), :],
                send_sem=so_send, recv_sem=so_recv,
                device_id=(target,), device_id_type=pl.DeviceIdType.MESH,
            )
            rq.start()
            ro.start()
            return rq, ro

        def wait_chunk(origin, sq_send, sq_recv, so_send, so_recv):
            rq = pltpu.make_async_remote_copy(
                src_ref=wq_buf.at[:, pl.ds(origin * CHUNK, CHUNK)],
                dst_ref=wq_buf.at[:, pl.ds(origin * CHUNK, CHUNK)],
                send_sem=sq_send, recv_sem=sq_recv,
                device_id=(left,), device_id_type=pl.DeviceIdType.MESH,
            )
            ro = pltpu.make_async_remote_copy(
                src_ref=wo_buf.at[pl.ds(origin * CHUNK, CHUNK), :],
                dst_ref=wo_buf.at[pl.ds(origin * CHUNK, CHUNK), :],
                send_sem=so_send, recv_sem=so_recv,
                device_id=(left,), device_id_type=pl.DeviceIdType.MESH,
            )
            rq.wait_recv()
            ro.wait_recv()

        def process_chunk(j, first):
            for b in range(n_b):
                q_c = lax.dot_general(
                    x_ref[b], wq_buf[:, pl.ds(j * CHUNK, CHUNK)],
                    (((1,), (0,)), ((), ())),
                    preferred_element_type=F32,
                ).astype(BF16)
                for hh in range(H_PER):
                    head = j * H_PER + hh
                    q_h = q_c[:, hh * DH:(hh + 1) * DH]
                    k_h = k_ref[b, head]
                    s = lax.dot_general(
                        q_h, k_h, (((1,), (1,)), ((), ())),
                        preferred_element_type=F32,
                    ) * 0.125 + neg
                    m = s.max(axis=-1, keepdims=True)
                    w = jnp.exp(s - m)
                    w = (w / w.sum(axis=-1, keepdims=True)).astype(BF16)
                    c = lax.dot_general(
                        w, v_ref[b, head], (((1,), (0,)), ((), ())),
                        preferred_element_type=F32,
                    )
                    ctx_buf[:, pl.ds(hh * DH, DH)] = c.astype(BF16)
                contrib = lax.dot_general(
                    ctx_buf[...], wo_buf[pl.ds(j * CHUNK, CHUNK), :],
                    (((1,), (0,)), ((), ())),
                    preferred_element_type=F32,
                )
                if first:
                    out_ref[b] = contrib
                else:
                    out_ref[b] = out_ref[b] + contrib

        sends = []
        sends.extend(send_chunk(my, right, cw_send_q.at[0], cw_recv_q.at[0],
                                cw_send_o.at[0], cw_recv_o.at[0]))
        sends.extend(send_chunk(my, left, ccw_send_q.at[0], ccw_recv_q.at[0],
                                ccw_send_o.at[0], ccw_recv_o.at[0]))
        process_chunk(my, first=True)

        for h in range(N_CW):
            origin = lax.rem(my - 1 - h + N_DEV, N_DEV)
            wait_chunk(origin, cw_send_q.at[h], cw_recv_q.at[h],
                       cw_send_o.at[h], cw_recv_o.at[h])
            if h + 1 < N_CW:
                sends.extend(send_chunk(
                    origin, right, cw_send_q.at[h + 1], cw_recv_q.at[h + 1],
                    cw_send_o.at[h + 1], cw_recv_o.at[h + 1]))
            process_chunk(origin, first=False)

            if h < N_CCW:
                origin = lax.rem(my + 1 + h, N_DEV)
                wait_chunk(origin, ccw_send_q.at[h], ccw_recv_q.at[h],
                           ccw_send_o.at[h], ccw_recv_o.at[h])
                if h + 1 < N_CCW:
                    sends.extend(send_chunk(
                        origin, left,
                        ccw_send_q.at[h + 1], ccw_recv_q.at[h + 1],
                        ccw_send_o.at[h + 1], ccw_recv_o.at[h + 1]))
                process_chunk(origin, first=False)

        for r in sends:
            r.wait_send()

    return pl.pallas_call(
        body,
        out_shape=jax.ShapeDtypeStruct((n_b, SQ, D_MODEL), F32),
        in_specs=[pl.BlockSpec(memory_space=pltpu.VMEM)] * 5,
        out_specs=pl.BlockSpec(memory_space=pltpu.VMEM),
        scratch_shapes=[
            pltpu.VMEM((D_MODEL, D_HIDDEN), BF16),
            pltpu.VMEM((D_HIDDEN, D_MODEL), BF16),
            pltpu.VMEM((SQ, CHUNK), BF16),
            pltpu.SemaphoreType.DMA((N_CW,)),
            pltpu.SemaphoreType.DMA((N_CW,)),
            pltpu.SemaphoreType.DMA((N_CW,)),
            pltpu.SemaphoreType.DMA((N_CW,)),
            pltpu.SemaphoreType.DMA((N_CCW,)),
            pltpu.SemaphoreType.DMA((N_CCW,)),
            pltpu.SemaphoreType.DMA((N_CCW,)),
            pltpu.SemaphoreType.DMA((N_CCW,)),
        ],
        compiler_params=pltpu.CompilerParams(collective_id=0),
    )(x_bf, wq_sh, wo_sh, k_bf, v_bf)


def kernel(x, Wq, K_ext, V_ext, Wo):
    my = lax.axis_index("i")
    b = x.shape[0]

    K_loc = lax.dynamic_slice_in_dim(K_ext, my * b, b, axis=0)
    V_loc = lax.dynamic_slice_in_dim(V_ext, my * b, b, axis=0)
    k_bf = K_loc.transpose(0, 2, 1, 3).astype(BF16)
    v_bf = V_loc.transpose(0, 2, 1, 3).astype(BF16)

    return _fused(
        x.astype(BF16),
        Wq.astype(BF16),
        Wo.astype(BF16),
        k_bf,
        v_bf,
    )
